# Initial kernel scaffold; baseline (speedup 1.0000x reference)
import jax
import jax.numpy as jnp
from jax.experimental import pallas as pl


def kernel(x, edge_index, sw1, sb1, bw1, bb1, sw2, sb2, bw2, bb2, sw3, sb3, bw3, bb3):
    raise NotImplementedError("write your pallas kernel here")


# speedup vs baseline: 8.8231x; 8.8231x over previous
import functools

import jax
import jax.numpy as jnp
from jax import lax
from jax.experimental import pallas as pl
from jax.experimental.pallas import tpu as pltpu
from jax.experimental.pallas import tpu_sc as plsc

N = 10000
E = 320000
D = 128
NC = 2
NS = 16
NW = NC * NS
EPW = E // NW
CH = 128
NFULL = EPW // CH
REM = EPW - NFULL * CH
NP = 10240
RPT = NP // NS


def _sc_agg_body(with_counts, *refs):
    if with_counts:
        (h_hbm, src_hbm, dst_hbm, zeros_hbm, zeros1_hbm, ones1_hbm,
         out_hbm, cnt_hbm,
         src_v, dst_v, rows_v, src_r, dst_r, rows_r, ones_v, ones_r,
         acc_sh, cnt_sh, sem) = refs
    else:
        (h_hbm, src_hbm, dst_hbm, zeros_hbm,
         out_hbm,
         src_v, dst_v, rows_v, src_r, dst_r, rows_r,
         acc_sh, sem) = refs

    c = lax.axis_index("c")
    s = lax.axis_index("s")
    wid = s * NC + c

    r0 = s * RPT
    pltpu.sync_copy(zeros_hbm.at[pl.ds(r0, RPT)], acc_sh.at[pl.ds(r0, RPT)])
    if with_counts:
        pltpu.sync_copy(zeros1_hbm.at[pl.ds(r0, RPT)], cnt_sh.at[pl.ds(r0, RPT)])
        pltpu.sync_copy(ones1_hbm, ones_v)
        pltpu.sync_copy(ones1_hbm.at[pl.ds(0, REM)], ones_r)
    plsc.subcore_barrier()

    base = wid * EPW

    def chunk(i, carry):
        off = pl.multiple_of(base + i * CH, CH)
        pltpu.sync_copy(src_hbm.at[pl.ds(off, CH)], src_v)
        pltpu.sync_copy(dst_hbm.at[pl.ds(off, CH)], dst_v)
        pltpu.async_copy(h_hbm.at[src_v], rows_v, sem).wait()
        pltpu.sync_copy(rows_v, acc_sh.at[dst_v], add=True)
        if with_counts:
            pltpu.sync_copy(ones_v, cnt_sh.at[dst_v], add=True)
        return carry

    lax.fori_loop(0, NFULL, chunk, 0)

    off = base + NFULL * CH
    pltpu.sync_copy(src_hbm.at[pl.ds(off, REM)], src_r)
    pltpu.sync_copy(dst_hbm.at[pl.ds(off, REM)], dst_r)
    pltpu.async_copy(h_hbm.at[src_r], rows_r, sem).wait()
    pltpu.sync_copy(rows_r, acc_sh.at[dst_r], add=True)
    if with_counts:
        pltpu.sync_copy(ones_r, cnt_sh.at[dst_r], add=True)

    plsc.subcore_barrier()

    pltpu.sync_copy(acc_sh.at[pl.ds(r0, RPT)], out_hbm.at[c, pl.ds(r0, RPT)])
    if with_counts:
        pltpu.sync_copy(cnt_sh.at[pl.ds(r0, RPT)],
                        cnt_hbm.at[pl.ds(c * NP + r0, RPT)])


def _make_sc_agg(with_counts):
    mesh = plsc.VectorSubcoreMesh(core_axis_name="c", subcore_axis_name="s")
    out_type = [jax.ShapeDtypeStruct((NC, NP, D), jnp.float32)]
    if with_counts:
        out_type.append(jax.ShapeDtypeStruct((NC * NP,), jnp.float32))
    scratch = [
        pltpu.VMEM((CH,), jnp.int32),
        pltpu.VMEM((CH,), jnp.int32),
        pltpu.VMEM((CH, D), jnp.float32),
        pltpu.VMEM((REM,), jnp.int32),
        pltpu.VMEM((REM,), jnp.int32),
        pltpu.VMEM((REM, D), jnp.float32),
    ]
    if with_counts:
        scratch += [
            pltpu.VMEM((CH,), jnp.float32),
            pltpu.VMEM((REM,), jnp.float32),
        ]
    scratch += [pltpu.VMEM_SHARED((NP, D), jnp.float32)]
    if with_counts:
        scratch += [pltpu.VMEM_SHARED((NP,), jnp.float32)]
    scratch += [pltpu.SemaphoreType.DMA]
    return pl.kernel(
        functools.partial(_sc_agg_body, with_counts),
        out_type=tuple(out_type) if with_counts else out_type[0],
        mesh=mesh,
        scratch_types=scratch,
    )


def _fastkan_block(mean, swa, swb, bwt, bias):
    ta = (mean + 1.0) * 0.5
    tb = (mean - 1.0) * 0.5
    ba = jnp.exp(-(ta * ta))
    bb = jnp.exp(-(tb * tb))
    sil = mean / (1.0 + jnp.exp(-mean))
    h = jnp.dot(ba, swa, preferred_element_type=jnp.float32)
    h += jnp.dot(bb, swb, preferred_element_type=jnp.float32)
    h += jnp.dot(sil, bwt, preferred_element_type=jnp.float32)
    return h + bias


def _layernorm_block(h):
    mu = jnp.mean(h, axis=-1, keepdims=True)
    var = jnp.mean((h - mu) ** 2, axis=-1, keepdims=True)
    return (h - mu) * lax.rsqrt(var + 1e-5)


def _tc_layer1_body(p0, p1, c0, c1, x, swa, swb, bwt, bias, out):
    agg = p0[...] + p1[...] + x[...]
    cnt = c0[...] + c1[...] + 1.0
    mean = agg / cnt
    h = _fastkan_block(mean, swa[...], swb[...], bwt[...], bias[...])
    out[...] = _layernorm_block(h)


def _tc_layer23_body(p0, p1, c0, c1, h1, swa2, swb2, bwt2, b2,
                     swa3, swb3, bwt3, b3, out):
    agg = p0[...] + p1[...] + h1[...]
    cnt = c0[...] + c1[...] + 1.0
    mean = agg / cnt
    h2 = _layernorm_block(
        _fastkan_block(mean, swa2[...], swb2[...], bwt2[...], b2[...]))
    o = _fastkan_block(h2, swa3[...], swb3[...], bwt3[...], b3[...])
    m = jnp.max(o, axis=-1, keepdims=True)
    lse = m + jnp.log(jnp.sum(jnp.exp(o - m), axis=-1, keepdims=True))
    out[...] = o - lse


_R = 1000


def _row_spec():
    return pl.BlockSpec((_R, D), lambda i: (i, 0))


def _cnt_spec():
    return pl.BlockSpec((_R, 1), lambda i: (i, 0))


def _w_spec():
    return pl.BlockSpec((D, D), lambda i: (0, 0))


def _b_spec():
    return pl.BlockSpec((1, D), lambda i: (0, 0))


def _split_w(sw, bw, sb, bb):
    swa = jnp.transpose(sw[:, 0::2])
    swb = jnp.transpose(sw[:, 1::2])
    bwt = jnp.transpose(bw)
    bias = (sb + bb).reshape(1, D)
    return swa, swb, bwt, bias


def kernel(x, edge_index, sw1, sb1, bw1, bb1, sw2, sb2, bw2, bb2,
           sw3, sb3, bw3, bb3):
    src = edge_index[0]
    dst = edge_index[1]
    zeros = jnp.zeros((NP, D), jnp.float32)
    zeros1 = jnp.zeros((NP,), jnp.float32)
    ones1 = jnp.ones((CH,), jnp.float32)

    w1 = _split_w(sw1, bw1, sb1, bb1)
    w2 = _split_w(sw2, bw2, sb2, bb2)
    w3 = _split_w(sw3, bw3, sb3, bb3)

    agg1, cnt1d = _make_sc_agg(True)(x, src, dst, zeros, zeros1, ones1)
    c0 = cnt1d[:N].reshape(N, 1)
    c1 = cnt1d[NP:NP + N].reshape(N, 1)

    tc1 = pl.pallas_call(
        _tc_layer1_body,
        grid=(N // _R,),
        in_specs=[_row_spec(), _row_spec(), _cnt_spec(), _cnt_spec(),
                  _row_spec(), _w_spec(), _w_spec(), _w_spec(), _b_spec()],
        out_specs=_row_spec(),
        out_shape=jax.ShapeDtypeStruct((N, D), jnp.float32),
    )
    h1 = tc1(agg1[0], agg1[1], c0, c1, x, *w1)

    agg2 = _make_sc_agg(False)(h1, src, dst, zeros)

    tc23 = pl.pallas_call(
        _tc_layer23_body,
        grid=(N // _R,),
        in_specs=[_row_spec(), _row_spec(), _cnt_spec(), _cnt_spec(),
                  _row_spec(),
                  _w_spec(), _w_spec(), _w_spec(), _b_spec(),
                  _w_spec(), _w_spec(), _w_spec(), _b_spec()],
        out_specs=_row_spec(),
        out_shape=jax.ShapeDtypeStruct((N, D), jnp.float32),
    )
    return tc23(agg2[0], agg2[1], c0, c1, h1, *w2, *w3)
